# Initial kernel scaffold; baseline (speedup 1.0000x reference)
import jax
import jax.numpy as jnp
from jax.experimental import pallas as pl


def kernel(x, target):
    raise NotImplementedError("write your pallas kernel here")


# speedup vs baseline: 3.3114x; 3.3114x over previous
import functools
import math

import jax
import jax.numpy as jnp
from jax import lax
from jax.experimental import pallas as pl
from jax.experimental.pallas import tpu as pltpu
from jax.experimental.pallas import tpu_sc as plsc

SIZE = 16384
ROWS = 4096
PADDING_IDX = 0
SMOOTHING = 0.1
CONFIDENCE = 1.0 - SMOOTHING
SMOOTH_VAL = SMOOTHING / (SIZE - 2)
ROW_CONST = (SIZE - 2) * SMOOTH_VAL * math.log(SMOOTH_VAL) + CONFIDENCE * math.log(CONFIDENCE)

ROW_BLOCK = 128
NUM_BLOCKS = ROWS // ROW_BLOCK


def _tc_body(t_ref, x_ref, out_ref):
    i = pl.program_id(0)

    @pl.when(i == 0)
    def _init():
        out_ref[0, 0] = 0.0

    xb = x_ref[...]
    m = (t_ref[0] != PADDING_IDX).astype(jnp.float32)
    mc = m.reshape(ROW_BLOCK, 1)
    xm = xb * mc
    p = jnp.sum(xm) - jnp.sum(xm[:, 0:1])
    out_ref[0, 0] += p


_tc_call = pl.pallas_call(
    _tc_body,
    grid=(NUM_BLOCKS,),
    in_specs=[
        pl.BlockSpec((1, 1, ROW_BLOCK), lambda i: (i, 0, 0)),
        pl.BlockSpec((ROW_BLOCK, SIZE), lambda i: (i, 0)),
    ],
    out_specs=pl.BlockSpec(memory_space=pltpu.SMEM),
    out_shape=jax.ShapeDtypeStruct((1, 1), jnp.float32),
    compiler_params=pltpu.CompilerParams(
        dimension_semantics=("arbitrary",),
    ),
)


_info = plsc.get_sparse_core_info()
_NC, _NS, _L = _info.num_cores, _info.num_subcores, _info.num_lanes
_NW = _NC * _NS
_CHUNK = ROWS // _NW

_sc_mesh = plsc.VectorSubcoreMesh(core_axis_name="c", subcore_axis_name="s")


@functools.partial(
    pl.kernel,
    mesh=_sc_mesh,
    out_type=jax.ShapeDtypeStruct((_NW, 2, _L), jnp.float32),
    scratch_types=[
        pltpu.VMEM((_CHUNK,), jnp.int32),
        pltpu.VMEM((_CHUNK,), jnp.int32),
        pltpu.VMEM((_CHUNK,), jnp.float32),
        pltpu.VMEM((2, _L), jnp.float32),
        pltpu.SemaphoreType.DMA,
    ],
)
def _sc_gather(xflat_hbm, t_hbm, out_hbm, t_v, idx_v, vals_v, acc_v, sem):
    wid = lax.axis_index("s") * _NC + lax.axis_index("c")
    base = wid * _CHUNK
    pltpu.sync_copy(t_hbm.at[pl.ds(base, _CHUNK)], t_v)
    lane = lax.iota(jnp.int32, _L)
    for j in range(_CHUNK // _L):
        t16 = t_v[pl.ds(j * _L, _L)]
        idx_v[pl.ds(j * _L, _L)] = (lane + (base + j * _L)) * SIZE + t16
    pltpu.async_copy(xflat_hbm.at[idx_v], vals_v, sem).wait()
    acc = jnp.zeros((_L,), jnp.float32)
    cnt = jnp.zeros((_L,), jnp.float32)
    for j in range(_CHUNK // _L):
        t16 = t_v[pl.ds(j * _L, _L)]
        m = t16 != PADDING_IDX
        acc = acc + jnp.where(m, vals_v[pl.ds(j * _L, _L)], 0.0)
        cnt = cnt + jnp.where(m, 1.0, 0.0)
    acc_v[0, :] = acc
    acc_v[1, :] = cnt
    pltpu.sync_copy(acc_v, out_hbm.at[wid])


def kernel(x, target):
    t32 = target.astype(jnp.int32)
    p = _tc_call(t32.reshape(NUM_BLOCKS, 1, ROW_BLOCK), x)
    parts = _sc_gather(x.reshape(ROWS * SIZE), t32)
    g = jnp.sum(parts[:, 0, :])
    nvalid = jnp.sum(parts[:, 1, :])
    return (nvalid * jnp.float32(ROW_CONST)
            - jnp.float32(SMOOTH_VAL) * p[0, 0]
            - jnp.float32(CONFIDENCE - SMOOTH_VAL) * g)


# speedup vs baseline: 8.9317x; 2.6973x over previous
import functools
import math

import jax
import jax.numpy as jnp
from jax import lax
from jax.experimental import pallas as pl
from jax.experimental.pallas import tpu as pltpu
from jax.experimental.pallas import tpu_sc as plsc

SIZE = 16384
ROWS = 4096
PADDING_IDX = 0
SMOOTHING = 0.1
CONFIDENCE = 1.0 - SMOOTHING
SMOOTH_VAL = SMOOTHING / (SIZE - 2)
ROW_CONST = (SIZE - 2) * SMOOTH_VAL * math.log(SMOOTH_VAL) + CONFIDENCE * math.log(CONFIDENCE)

ROW_BLOCK = 256
NUM_BLOCKS = ROWS // ROW_BLOCK


TC_GATHER_BLOCKS = NUM_BLOCKS // 2


def _tc_body(t_ref, x_ref, out_ref, g_ref, n_ref):
    i = pl.program_id(0)

    @pl.when(i == 0)
    def _init():
        out_ref[0, 0] = 0.0
        g_ref[0, 0] = 0.0
        n_ref[0, 0] = 0.0

    xb = x_ref[...]
    rs = jnp.sum(xb, axis=1)
    t = t_ref[0].reshape(ROW_BLOCK)
    valid = t != PADDING_IDX
    p = jnp.sum(jnp.where(valid, rs - xb[:, 0], 0.0))
    out_ref[0, 0] += p

    @pl.when(i < TC_GATHER_BLOCKS)
    def _gather():
        tcol = jnp.where(valid, t, -1).reshape(ROW_BLOCK, 1)
        colid = lax.broadcasted_iota(jnp.int32, (ROW_BLOCK, SIZE), 1)
        g = jnp.sum(jnp.where(colid == tcol, xb, 0.0))
        g_ref[0, 0] += g
        n_ref[0, 0] += jnp.sum(jnp.where(valid, 1.0, 0.0))


_tc_call = pl.pallas_call(
    _tc_body,
    grid=(NUM_BLOCKS,),
    in_specs=[
        pl.BlockSpec((1, 1, ROW_BLOCK), lambda i: (i, 0, 0)),
        pl.BlockSpec((ROW_BLOCK, SIZE), lambda i: (i, 0)),
    ],
    out_specs=[
        pl.BlockSpec(memory_space=pltpu.SMEM),
        pl.BlockSpec(memory_space=pltpu.SMEM),
        pl.BlockSpec(memory_space=pltpu.SMEM),
    ],
    out_shape=[
        jax.ShapeDtypeStruct((1, 1), jnp.float32),
        jax.ShapeDtypeStruct((1, 1), jnp.float32),
        jax.ShapeDtypeStruct((1, 1), jnp.float32),
    ],
    compiler_params=pltpu.CompilerParams(
        dimension_semantics=("arbitrary",),
    ),
)


_info = plsc.get_sparse_core_info()
_NC, _NS, _L = _info.num_cores, _info.num_subcores, _info.num_lanes
_NW = _NC * _NS
_SC_ROW0 = ROWS // 2
_CHUNK = (ROWS - _SC_ROW0) // _NW

_sc_mesh = plsc.VectorSubcoreMesh(core_axis_name="c", subcore_axis_name="s")


@functools.partial(
    pl.kernel,
    mesh=_sc_mesh,
    out_type=jax.ShapeDtypeStruct((_NW, 2, _L), jnp.float32),
    scratch_types=[
        pltpu.VMEM((_CHUNK,), jnp.int32),
        pltpu.VMEM((_L, 8, 128), jnp.float32),
        pltpu.VMEM((2, _L), jnp.float32),
        pltpu.SemaphoreType.DMA,
    ],
    compiler_params=pltpu.CompilerParams(use_tc_tiling_on_sc=True),
)
def _sc_gather(x_hbm, t_hbm, out_hbm, t_v, tiles_v, acc_v, sem):
    wid = lax.axis_index("s") * _NC + lax.axis_index("c")
    base = _SC_ROW0 + wid * _CHUNK
    pltpu.sync_copy(t_hbm.at[pl.ds(base, _CHUNK)], t_v)
    lane = lax.iota(jnp.int32, _L)
    acc = jnp.zeros((_L,), jnp.float32)
    cnt = jnp.zeros((_L,), jnp.float32)
    for r in range(_CHUNK // _L):
        t16 = t_v[pl.ds(r * _L, _L)]
        for sb in range(4):
            copies = []
            for k in range(sb * 4, sb * 4 + 4):
                col = pl.multiple_of(lax.bitwise_and(t16[k], ~127), 128)
                row = base + r * _L + (k & ~7)
                copies.append(pltpu.make_async_copy(
                    x_hbm.at[pl.ds(row, 8), pl.ds(col, 128)],
                    tiles_v.at[k], sem))
            for c in copies:
                c.start()
            for c in copies:
                c.wait()
        cnt = cnt + jnp.where(t16 != PADDING_IDX, 1.0, 0.0)
        for k in range(_L):
            tk = t16[k]
            c7 = jnp.where(tk != PADDING_IDX, lax.bitwise_and(tk, 127),
                           jnp.int32(-1024))
            for c in range(8):
                seg = tiles_v[k, k & 7, pl.ds(c * _L, _L)]
                acc = acc + jnp.where(lane == (c7 - c * _L), seg, 0.0)
    acc_v[0, :] = acc
    acc_v[1, :] = cnt
    pltpu.sync_copy(acc_v, out_hbm.at[wid])


def kernel(x, target):
    t32 = target.astype(jnp.int32)
    p, g_tc, n_tc = _tc_call(t32.reshape(NUM_BLOCKS, 1, ROW_BLOCK), x)
    parts = _sc_gather(x, t32)
    g = jnp.sum(parts[:, 0, :]) + g_tc[0, 0]
    nvalid = jnp.sum(parts[:, 1, :]) + n_tc[0, 0]
    return (nvalid * jnp.float32(ROW_CONST)
            - jnp.float32(SMOOTH_VAL) * p[0, 0]
            - jnp.float32(CONFIDENCE - SMOOTH_VAL) * g)


# speedup vs baseline: 9.5581x; 1.0701x over previous
import functools
import math

import jax
import jax.numpy as jnp
from jax import lax
from jax.experimental import pallas as pl
from jax.experimental.pallas import tpu as pltpu
from jax.experimental.pallas import tpu_sc as plsc

SIZE = 16384
ROWS = 4096
PADDING_IDX = 0
SMOOTHING = 0.1
CONFIDENCE = 1.0 - SMOOTHING
SMOOTH_VAL = SMOOTHING / (SIZE - 2)
ROW_CONST = (SIZE - 2) * SMOOTH_VAL * math.log(SMOOTH_VAL) + CONFIDENCE * math.log(CONFIDENCE)

ROW_BLOCK = 256
NUM_BLOCKS = ROWS // ROW_BLOCK


def _tc_body(t_ref, x_ref, out_ref):
    i = pl.program_id(0)

    @pl.when(i == 0)
    def _init():
        out_ref[0, 0] = 0.0

    xb = x_ref[...]
    rs = jnp.sum(xb, axis=1)
    t = t_ref[0].reshape(ROW_BLOCK)
    p = jnp.sum(jnp.where(t != PADDING_IDX, rs - xb[:, 0], 0.0))
    out_ref[0, 0] += p


_tc_call = pl.pallas_call(
    _tc_body,
    grid=(NUM_BLOCKS,),
    in_specs=[
        pl.BlockSpec((1, 1, ROW_BLOCK), lambda i: (i, 0, 0)),
        pl.BlockSpec((ROW_BLOCK, SIZE), lambda i: (i, 0)),
    ],
    out_specs=pl.BlockSpec(memory_space=pltpu.SMEM),
    out_shape=jax.ShapeDtypeStruct((1, 1), jnp.float32),
    compiler_params=pltpu.CompilerParams(
        dimension_semantics=("arbitrary",),
    ),
)


_info = plsc.get_sparse_core_info()
_NC, _NS, _L = _info.num_cores, _info.num_subcores, _info.num_lanes
_NW = _NC * _NS
_CHUNK = ROWS // _NW

_sc_mesh = plsc.VectorSubcoreMesh(core_axis_name="c", subcore_axis_name="s")


@functools.partial(
    pl.kernel,
    mesh=_sc_mesh,
    out_type=jax.ShapeDtypeStruct((_NW, 2, _L), jnp.float32),
    scratch_types=[
        pltpu.VMEM((_CHUNK,), jnp.int32),
        pltpu.VMEM((_L, 8, 128), jnp.float32),
        pltpu.VMEM((2, _L), jnp.float32),
        pltpu.SemaphoreType.DMA,
    ],
    compiler_params=pltpu.CompilerParams(use_tc_tiling_on_sc=True),
)
def _sc_gather(x_hbm, t_hbm, out_hbm, t_v, tiles_v, acc_v, sem):
    wid = lax.axis_index("s") * _NC + lax.axis_index("c")
    base = wid * _CHUNK
    pltpu.sync_copy(t_hbm.at[pl.ds(base, _CHUNK)], t_v)
    lane = lax.iota(jnp.int32, _L)
    acc = jnp.zeros((_L,), jnp.float32)
    cnt = jnp.zeros((_L,), jnp.float32)
    for r in range(_CHUNK // _L):
        t16 = t_v[pl.ds(r * _L, _L)]
        for sb in range(4):
            copies = []
            for k in range(sb * 4, sb * 4 + 4):
                col = pl.multiple_of(lax.bitwise_and(t16[k], ~127), 128)
                row = base + r * _L + (k & ~7)
                copies.append(pltpu.make_async_copy(
                    x_hbm.at[pl.ds(row, 8), pl.ds(col, 128)],
                    tiles_v.at[k], sem))
            for c in copies:
                c.start()
            for c in copies:
                c.wait()
        cnt = cnt + jnp.where(t16 != PADDING_IDX, 1.0, 0.0)
        for k in range(_L):
            tk = t16[k]
            c7 = jnp.where(tk != PADDING_IDX, lax.bitwise_and(tk, 127),
                           jnp.int32(-1024))
            for c in range(8):
                seg = tiles_v[k, k & 7, pl.ds(c * _L, _L)]
                acc = acc + jnp.where(lane == (c7 - c * _L), seg, 0.0)
    acc_v[0, :] = acc
    acc_v[1, :] = cnt
    pltpu.sync_copy(acc_v, out_hbm.at[wid])


def kernel(x, target):
    t32 = target.astype(jnp.int32)
    p = _tc_call(t32.reshape(NUM_BLOCKS, 1, ROW_BLOCK), x)
    parts = _sc_gather(x, t32)
    g = jnp.sum(parts[:, 0, :])
    nvalid = jnp.sum(parts[:, 1, :])
    return (nvalid * jnp.float32(ROW_CONST)
            - jnp.float32(SMOOTH_VAL) * p[0, 0]
            - jnp.float32(CONFIDENCE - SMOOTH_VAL) * g)
